# Initial kernel scaffold; baseline (speedup 1.0000x reference)
import jax
import jax.numpy as jnp
from jax.experimental import pallas as pl


def kernel(x, edge_index, W0, b0, W1, b1, W2, b2, Wl, bl):
    raise NotImplementedError("write your pallas kernel here")


# speedup vs baseline: 7.1134x; 7.1134x over previous
import functools

import jax
import jax.numpy as jnp
from jax import lax
from jax.experimental import pallas as pl
from jax.experimental.pallas import tpu as pltpu
from jax.experimental.pallas import tpu_sc as plsc

N_NODES = 10000
N_EDGES = 320000
N_FEAT = 128
N_CLASSES = 40

NC = 2
NS = 16
NW = NC * NS

EPW = N_EDGES // NW
CH = 100
NCHUNK = EPW // CH
INNER = 4
PADR = 640
ACC_ROWS = NS * PADR

MB = 400
NBLK = N_NODES // MB


def _agg_body(h_hbm, src_hbm, dst_hbm, z_hbm, out_hbm,
              src_v, dst_v, rows_v, acc_sh, sem):
    c = lax.axis_index("c")
    s = lax.axis_index("s")
    wid = s * NC + c

    pltpu.sync_copy(z_hbm, rows_v)
    for k in range(PADR // CH):
        pltpu.sync_copy(rows_v, acc_sh.at[pl.ds(s * PADR + k * CH, CH)])
    rem = PADR - (PADR // CH) * CH
    if rem:
        pltpu.sync_copy(rows_v.at[pl.ds(0, rem)],
                        acc_sh.at[pl.ds(s * PADR + (PADR // CH) * CH, rem)])
    plsc.subcore_barrier()

    pltpu.sync_copy(src_hbm.at[wid], src_v)
    pltpu.sync_copy(dst_hbm.at[wid], dst_v)

    @pl.loop(0, NCHUNK // INNER)
    def _chunks(o):
        for j in range(INNER):
            k = o * INNER + j
            pltpu.async_copy(h_hbm.at[src_v.at[k]], rows_v, sem).wait()
            pltpu.sync_copy(rows_v, acc_sh.at[dst_v.at[k]], add=True)

    plsc.subcore_barrier()

    pltpu.sync_copy(acc_sh.at[pl.ds(s * PADR, PADR)], out_hbm.at[c, s])


@jax.jit
def _aggregate(h, src3d, dst3d, zeros):
    mesh = plsc.VectorSubcoreMesh(core_axis_name="c", subcore_axis_name="s",
                                  num_cores=NC, num_subcores=NS)
    return pl.kernel(
        _agg_body,
        out_type=jax.ShapeDtypeStruct((NC, NS, PADR, N_FEAT), jnp.float32),
        mesh=mesh,
        scratch_types=[
            pltpu.VMEM((NCHUNK, CH), jnp.int32),
            pltpu.VMEM((NCHUNK, CH), jnp.int32),
            pltpu.VMEM((CH, N_FEAT), jnp.float32),
            pltpu.VMEM_SHARED((ACC_ROWS, N_FEAT), jnp.float32),
            pltpu.SemaphoreType.DMA,
        ],
    )(h, src3d, dst3d, zeros)


def _mm_body(x_ref, w_ref, b_ref, o_ref):
    o_ref[...] = jnp.dot(x_ref[...], w_ref[...],
                         preferred_element_type=jnp.float32) + b_ref[...]


def _fuse_body(p0_ref, p1_ref, w_ref, b_ref, h_ref, t_ref):
    h = jax.nn.relu(p0_ref[...] + p1_ref[...])
    h_ref[...] = h
    t_ref[...] = jnp.dot(h, w_ref[...],
                         preferred_element_type=jnp.float32) + b_ref[...]


def _fuse_res_body(p0_ref, p1_ref, r_ref, w_ref, b_ref, h_ref, t_ref):
    h = jax.nn.relu(p0_ref[...] + p1_ref[...]) + r_ref[...]
    h_ref[...] = h
    t_ref[...] = jnp.dot(h, w_ref[...],
                         preferred_element_type=jnp.float32) + b_ref[...]


def _final_body(p0_ref, p1_ref, r_ref, w_ref, b_ref, o_ref):
    h = jax.nn.relu(p0_ref[...] + p1_ref[...]) + r_ref[...]
    o_ref[...] = jnp.dot(h, w_ref[...],
                         preferred_element_type=jnp.float32) + b_ref[...]


_row_spec = pl.BlockSpec((MB, N_FEAT), lambda i: (i, 0))
_w_spec = pl.BlockSpec((N_FEAT, N_FEAT), lambda i: (0, 0))
_b_spec = pl.BlockSpec((1, N_FEAT), lambda i: (0, 0))


def _mm(x, w, b):
    return pl.pallas_call(
        _mm_body,
        grid=(NBLK,),
        in_specs=[_row_spec, _w_spec, _b_spec],
        out_specs=_row_spec,
        out_shape=jax.ShapeDtypeStruct((N_NODES, N_FEAT), jnp.float32),
    )(x, w, b)


def _fuse(p0, p1, w, b):
    return pl.pallas_call(
        _fuse_body,
        grid=(NBLK,),
        in_specs=[_row_spec, _row_spec, _w_spec, _b_spec],
        out_specs=[_row_spec, _row_spec],
        out_shape=[jax.ShapeDtypeStruct((N_NODES, N_FEAT), jnp.float32),
                   jax.ShapeDtypeStruct((N_NODES, N_FEAT), jnp.float32)],
    )(p0, p1, w, b)


def _fuse_res(p0, p1, r, w, b):
    return pl.pallas_call(
        _fuse_res_body,
        grid=(NBLK,),
        in_specs=[_row_spec, _row_spec, _row_spec, _w_spec, _b_spec],
        out_specs=[_row_spec, _row_spec],
        out_shape=[jax.ShapeDtypeStruct((N_NODES, N_FEAT), jnp.float32),
                   jax.ShapeDtypeStruct((N_NODES, N_FEAT), jnp.float32)],
    )(p0, p1, r, w, b)


def _final(p0, p1, r, w, b):
    return pl.pallas_call(
        _final_body,
        grid=(NBLK,),
        in_specs=[_row_spec, _row_spec, _row_spec, _w_spec, _b_spec],
        out_specs=_row_spec,
        out_shape=jax.ShapeDtypeStruct((N_NODES, N_FEAT), jnp.float32),
    )(p0, p1, r, w, b)


def kernel(x, edge_index, W0, b0, W1, b1, W2, b2, Wl, bl):
    src3d = edge_index[0].astype(jnp.int32).reshape(NW, NCHUNK, CH)
    dst3d = edge_index[1].astype(jnp.int32).reshape(NW, NCHUNK, CH)
    zeros = jnp.zeros((CH, N_FEAT), jnp.float32)
    wl_pad = jnp.zeros((N_FEAT, N_FEAT), jnp.float32).at[:, :N_CLASSES].set(Wl)
    bl_pad = jnp.zeros((N_FEAT,), jnp.float32).at[:N_CLASSES].set(bl)

    def parts(p):
        p = p.reshape(NC, ACC_ROWS, N_FEAT)
        return p[0, :N_NODES], p[1, :N_NODES]

    t0 = _mm(x, W0, b0.reshape(1, N_FEAT))
    p0, p1 = parts(_aggregate(t0, src3d, dst3d, zeros))
    h0, t1 = _fuse(p0, p1, W1, b1.reshape(1, N_FEAT))
    p0, p1 = parts(_aggregate(t1, src3d, dst3d, zeros))
    h1, t2 = _fuse_res(p0, p1, h0, W2, b2.reshape(1, N_FEAT))
    p0, p1 = parts(_aggregate(t2, src3d, dst3d, zeros))
    out_pad = _final(p0, p1, h1, wl_pad, bl_pad.reshape(1, N_FEAT))
    return out_pad[:, :N_CLASSES]


# speedup vs baseline: 10.4935x; 1.4752x over previous
import functools

import jax
import jax.numpy as jnp
from jax import lax
from jax.experimental import pallas as pl
from jax.experimental.pallas import tpu as pltpu
from jax.experimental.pallas import tpu_sc as plsc

N_NODES = 10000
N_EDGES = 320000
N_FEAT = 128
N_CLASSES = 40

NC = 2
NS = 16
NW = NC * NS

EPW = N_EDGES // NW
CH = 100
NCHUNK = EPW // CH
HALF = NCHUNK // 2
PADR = 640
ACC_ROWS = NS * PADR

MB = 400
NBLK = N_NODES // MB


def _agg_body(h_hbm, src_hbm, dst_hbm, z_hbm, out_hbm,
              src_v, dst_v, rows_v, acc_sh, gsem, ssem):
    c = lax.axis_index("c")
    s = lax.axis_index("s")
    wid = s * NC + c

    pltpu.sync_copy(z_hbm, rows_v.at[0])
    for k in range(PADR // CH):
        pltpu.sync_copy(rows_v.at[0], acc_sh.at[pl.ds(s * PADR + k * CH, CH)])
    rem = PADR - (PADR // CH) * CH
    if rem:
        pltpu.sync_copy(rows_v.at[0].at[pl.ds(0, rem)],
                        acc_sh.at[pl.ds(s * PADR + (PADR // CH) * CH, rem)])
    plsc.subcore_barrier()

    for hf in range(NCHUNK // HALF):
        pltpu.sync_copy(src_hbm.at[wid, hf], src_v)
        pltpu.sync_copy(dst_hbm.at[wid, hf], dst_v)
        pltpu.async_copy(h_hbm.at[src_v.at[0]], rows_v.at[0], gsem)

        @pl.loop(0, HALF // 2)
        def _chunks(o):
            for j in range(2):
                k = o * 2 + j
                nb = (j + 1) % 2

                @pl.when(k >= 1)
                def _drain_prev():
                    pltpu.make_async_copy(
                        rows_v.at[nb], acc_sh.at[dst_v.at[k - 1]], ssem).wait()

                @pl.when(k + 1 < HALF)
                def _fire_next():
                    pltpu.async_copy(h_hbm.at[src_v.at[k + 1]], rows_v.at[nb],
                                     gsem)

                pltpu.make_async_copy(h_hbm.at[src_v.at[k]], rows_v.at[j],
                                      gsem).wait()
                pltpu.async_copy(rows_v.at[j], acc_sh.at[dst_v.at[k]], ssem,
                                 add=True)

        pltpu.make_async_copy(rows_v.at[(HALF - 1) % 2],
                              acc_sh.at[dst_v.at[HALF - 1]], ssem).wait()

    plsc.subcore_barrier()

    pltpu.sync_copy(acc_sh.at[pl.ds(s * PADR, PADR)], out_hbm.at[c, s])


@jax.jit
def _aggregate(h, src3d, dst3d, zeros):
    mesh = plsc.VectorSubcoreMesh(core_axis_name="c", subcore_axis_name="s",
                                  num_cores=NC, num_subcores=NS)
    return pl.kernel(
        _agg_body,
        out_type=jax.ShapeDtypeStruct((NC, NS, PADR, N_FEAT), jnp.float32),
        mesh=mesh,
        scratch_types=[
            pltpu.VMEM((HALF, CH), jnp.int32),
            pltpu.VMEM((HALF, CH), jnp.int32),
            pltpu.VMEM((2, CH, N_FEAT), jnp.float32),
            pltpu.VMEM_SHARED((ACC_ROWS, N_FEAT), jnp.float32),
            pltpu.SemaphoreType.DMA,
            pltpu.SemaphoreType.DMA,
        ],
    )(h, src3d, dst3d, zeros)


def _mm_body(x_ref, w_ref, b_ref, o_ref):
    o_ref[...] = jnp.dot(x_ref[...], w_ref[...],
                         preferred_element_type=jnp.float32) + b_ref[...]


def _fuse_body(p0_ref, p1_ref, w_ref, b_ref, h_ref, t_ref):
    h = jax.nn.relu(p0_ref[...] + p1_ref[...])
    h_ref[...] = h
    t_ref[...] = jnp.dot(h, w_ref[...],
                         preferred_element_type=jnp.float32) + b_ref[...]


def _fuse_res_body(p0_ref, p1_ref, r_ref, w_ref, b_ref, h_ref, t_ref):
    h = jax.nn.relu(p0_ref[...] + p1_ref[...]) + r_ref[...]
    h_ref[...] = h
    t_ref[...] = jnp.dot(h, w_ref[...],
                         preferred_element_type=jnp.float32) + b_ref[...]


def _final_body(p0_ref, p1_ref, r_ref, w_ref, b_ref, o_ref):
    h = jax.nn.relu(p0_ref[...] + p1_ref[...]) + r_ref[...]
    o_ref[...] = jnp.dot(h, w_ref[...],
                         preferred_element_type=jnp.float32) + b_ref[...]


_row_spec = pl.BlockSpec((MB, N_FEAT), lambda i: (i, 0))
_w_spec = pl.BlockSpec((N_FEAT, N_FEAT), lambda i: (0, 0))
_b_spec = pl.BlockSpec((1, N_FEAT), lambda i: (0, 0))


def _mm(x, w, b):
    return pl.pallas_call(
        _mm_body,
        grid=(NBLK,),
        in_specs=[_row_spec, _w_spec, _b_spec],
        out_specs=_row_spec,
        out_shape=jax.ShapeDtypeStruct((N_NODES, N_FEAT), jnp.float32),
    )(x, w, b)


def _fuse(p0, p1, w, b):
    return pl.pallas_call(
        _fuse_body,
        grid=(NBLK,),
        in_specs=[_row_spec, _row_spec, _w_spec, _b_spec],
        out_specs=[_row_spec, _row_spec],
        out_shape=[jax.ShapeDtypeStruct((N_NODES, N_FEAT), jnp.float32),
                   jax.ShapeDtypeStruct((N_NODES, N_FEAT), jnp.float32)],
    )(p0, p1, w, b)


def _fuse_res(p0, p1, r, w, b):
    return pl.pallas_call(
        _fuse_res_body,
        grid=(NBLK,),
        in_specs=[_row_spec, _row_spec, _row_spec, _w_spec, _b_spec],
        out_specs=[_row_spec, _row_spec],
        out_shape=[jax.ShapeDtypeStruct((N_NODES, N_FEAT), jnp.float32),
                   jax.ShapeDtypeStruct((N_NODES, N_FEAT), jnp.float32)],
    )(p0, p1, r, w, b)


def _final(p0, p1, r, w, b):
    return pl.pallas_call(
        _final_body,
        grid=(NBLK,),
        in_specs=[_row_spec, _row_spec, _row_spec, _w_spec, _b_spec],
        out_specs=_row_spec,
        out_shape=jax.ShapeDtypeStruct((N_NODES, N_FEAT), jnp.float32),
    )(p0, p1, r, w, b)


def kernel(x, edge_index, W0, b0, W1, b1, W2, b2, Wl, bl):
    src3d = edge_index[0].astype(jnp.int32).reshape(NW, NCHUNK // HALF, HALF, CH)
    dst3d = edge_index[1].astype(jnp.int32).reshape(NW, NCHUNK // HALF, HALF, CH)
    zeros = jnp.zeros((CH, N_FEAT), jnp.float32)
    wl_pad = jnp.zeros((N_FEAT, N_FEAT), jnp.float32).at[:, :N_CLASSES].set(Wl)
    bl_pad = jnp.zeros((N_FEAT,), jnp.float32).at[:N_CLASSES].set(bl)

    def parts(p):
        p = p.reshape(NC, ACC_ROWS, N_FEAT)
        return p[0, :N_NODES], p[1, :N_NODES]

    t0 = _mm(x, W0, b0.reshape(1, N_FEAT))
    p0, p1 = parts(_aggregate(t0, src3d, dst3d, zeros))
    h0, t1 = _fuse(p0, p1, W1, b1.reshape(1, N_FEAT))
    p0, p1 = parts(_aggregate(t1, src3d, dst3d, zeros))
    h1, t2 = _fuse_res(p0, p1, h0, W2, b2.reshape(1, N_FEAT))
    p0, p1 = parts(_aggregate(t2, src3d, dst3d, zeros))
    out_pad = _final(p0, p1, h1, wl_pad, bl_pad.reshape(1, N_FEAT))
    return out_pad[:, :N_CLASSES]


# speedup vs baseline: 11.2433x; 1.0715x over previous
import functools

import jax
import jax.numpy as jnp
from jax import lax
from jax.experimental import pallas as pl
from jax.experimental.pallas import tpu as pltpu
from jax.experimental.pallas import tpu_sc as plsc

N_NODES = 10000
N_EDGES = 320000
N_FEAT = 128
N_CLASSES = 40

NC = 2
NS = 16
NW = NC * NS

EPW = N_EDGES // NW
CH = 100
NCHUNK = EPW // CH
NBUF = 2
STAGE = 50
STAGES = NCHUNK // STAGE
PADR = 640
ACC_ROWS = NS * PADR

MB = 1000
NBLK = N_NODES // MB


def _agg_body(h_hbm, src_hbm, dst_hbm, z_hbm, out_hbm,
              src_v, dst_v, rows_v, acc_sh, gsem, ssem):
    c = lax.axis_index("c")
    s = lax.axis_index("s")
    wid = s * NC + c

    pltpu.sync_copy(z_hbm, rows_v.at[0])
    for k in range(PADR // CH):
        pltpu.sync_copy(rows_v.at[0], acc_sh.at[pl.ds(s * PADR + k * CH, CH)])
    rem = PADR - (PADR // CH) * CH
    if rem:
        pltpu.sync_copy(rows_v.at[0].at[pl.ds(0, rem)],
                        acc_sh.at[pl.ds(s * PADR + (PADR // CH) * CH, rem)])
    plsc.subcore_barrier()

    for st in range(STAGES):
        pltpu.sync_copy(src_hbm.at[wid, st], src_v)
        pltpu.sync_copy(dst_hbm.at[wid, st], dst_v)
        pltpu.async_copy(h_hbm.at[src_v.at[0]], rows_v.at[0], gsem)

        @pl.loop(0, STAGE // NBUF)
        def _chunks(o):
            for j in range(NBUF):
                k = o * NBUF + j
                nb = (j + 1) % NBUF

                @pl.when(k >= 1)
                def _drain_prev():
                    pltpu.make_async_copy(
                        rows_v.at[nb], acc_sh.at[dst_v.at[k - 1]], ssem).wait()

                @pl.when(k + 1 < STAGE)
                def _fire_next():
                    pltpu.async_copy(h_hbm.at[src_v.at[k + 1]], rows_v.at[nb],
                                     gsem)

                pltpu.make_async_copy(h_hbm.at[src_v.at[k]], rows_v.at[j],
                                      gsem).wait()
                pltpu.async_copy(rows_v.at[j], acc_sh.at[dst_v.at[k]], ssem,
                                 add=True)

        pltpu.make_async_copy(rows_v.at[(STAGE - 1) % NBUF],
                              acc_sh.at[dst_v.at[STAGE - 1]], ssem).wait()

    plsc.subcore_barrier()

    pltpu.sync_copy(acc_sh.at[pl.ds(s * PADR, PADR)], out_hbm.at[c, s])


@jax.jit
def _aggregate(h, src3d, dst3d, zeros):
    mesh = plsc.VectorSubcoreMesh(core_axis_name="c", subcore_axis_name="s",
                                  num_cores=NC, num_subcores=NS)
    return pl.kernel(
        _agg_body,
        out_type=jax.ShapeDtypeStruct((NC, NS, PADR, N_FEAT), jnp.float32),
        mesh=mesh,
        scratch_types=[
            pltpu.VMEM((STAGE, CH), jnp.int32),
            pltpu.VMEM((STAGE, CH), jnp.int32),
            pltpu.VMEM((NBUF, CH, N_FEAT), jnp.float32),
            pltpu.VMEM_SHARED((ACC_ROWS, N_FEAT), jnp.float32),
            pltpu.SemaphoreType.DMA,
            pltpu.SemaphoreType.DMA,
        ],
    )(h, src3d, dst3d, zeros)


def _mm_body(x_ref, w_ref, b_ref, o_ref):
    o_ref[...] = jnp.dot(x_ref[...], w_ref[...],
                         preferred_element_type=jnp.float32) + b_ref[...]


def _fuse_body(p0_ref, p1_ref, w_ref, b_ref, h_ref, t_ref):
    h = jax.nn.relu(p0_ref[...] + p1_ref[...])
    h_ref[...] = h
    t_ref[...] = jnp.dot(h, w_ref[...],
                         preferred_element_type=jnp.float32) + b_ref[...]


def _fuse_res_body(p0_ref, p1_ref, r_ref, w_ref, b_ref, h_ref, t_ref):
    h = jax.nn.relu(p0_ref[...] + p1_ref[...]) + r_ref[...]
    h_ref[...] = h
    t_ref[...] = jnp.dot(h, w_ref[...],
                         preferred_element_type=jnp.float32) + b_ref[...]


def _final_body(p0_ref, p1_ref, r_ref, w_ref, b_ref, o_ref):
    h = jax.nn.relu(p0_ref[...] + p1_ref[...]) + r_ref[...]
    o_ref[...] = jnp.dot(h, w_ref[...],
                         preferred_element_type=jnp.float32) + b_ref[...]


_row_spec = pl.BlockSpec((MB, N_FEAT), lambda i: (i, 0))
_w_spec = pl.BlockSpec((N_FEAT, N_FEAT), lambda i: (0, 0))
_b_spec = pl.BlockSpec((1, N_FEAT), lambda i: (0, 0))


def _mm(x, w, b):
    return pl.pallas_call(
        _mm_body,
        grid=(NBLK,),
        in_specs=[_row_spec, _w_spec, _b_spec],
        out_specs=_row_spec,
        out_shape=jax.ShapeDtypeStruct((N_NODES, N_FEAT), jnp.float32),
    )(x, w, b)


def _fuse(p0, p1, w, b):
    return pl.pallas_call(
        _fuse_body,
        grid=(NBLK,),
        in_specs=[_row_spec, _row_spec, _w_spec, _b_spec],
        out_specs=[_row_spec, _row_spec],
        out_shape=[jax.ShapeDtypeStruct((N_NODES, N_FEAT), jnp.float32),
                   jax.ShapeDtypeStruct((N_NODES, N_FEAT), jnp.float32)],
    )(p0, p1, w, b)


def _fuse_res(p0, p1, r, w, b):
    return pl.pallas_call(
        _fuse_res_body,
        grid=(NBLK,),
        in_specs=[_row_spec, _row_spec, _row_spec, _w_spec, _b_spec],
        out_specs=[_row_spec, _row_spec],
        out_shape=[jax.ShapeDtypeStruct((N_NODES, N_FEAT), jnp.float32),
                   jax.ShapeDtypeStruct((N_NODES, N_FEAT), jnp.float32)],
    )(p0, p1, r, w, b)


def _final(p0, p1, r, w, b):
    return pl.pallas_call(
        _final_body,
        grid=(NBLK,),
        in_specs=[_row_spec, _row_spec, _row_spec, _w_spec, _b_spec],
        out_specs=_row_spec,
        out_shape=jax.ShapeDtypeStruct((N_NODES, N_FEAT), jnp.float32),
    )(p0, p1, r, w, b)


def kernel(x, edge_index, W0, b0, W1, b1, W2, b2, Wl, bl):
    src3d = edge_index[0].astype(jnp.int32).reshape(NW, STAGES, STAGE, CH)
    dst3d = edge_index[1].astype(jnp.int32).reshape(NW, STAGES, STAGE, CH)
    zeros = jnp.zeros((CH, N_FEAT), jnp.float32)
    wl_pad = jnp.zeros((N_FEAT, N_FEAT), jnp.float32).at[:, :N_CLASSES].set(Wl)
    bl_pad = jnp.zeros((N_FEAT,), jnp.float32).at[:N_CLASSES].set(bl)

    def parts(p):
        p = p.reshape(NC, ACC_ROWS, N_FEAT)
        return p[0, :N_NODES], p[1, :N_NODES]

    t0 = _mm(x, W0, b0.reshape(1, N_FEAT))
    p0, p1 = parts(_aggregate(t0, src3d, dst3d, zeros))
    h0, t1 = _fuse(p0, p1, W1, b1.reshape(1, N_FEAT))
    p0, p1 = parts(_aggregate(t1, src3d, dst3d, zeros))
    h1, t2 = _fuse_res(p0, p1, h0, W2, b2.reshape(1, N_FEAT))
    p0, p1 = parts(_aggregate(t2, src3d, dst3d, zeros))
    out_pad = _final(p0, p1, h1, wl_pad, bl_pad.reshape(1, N_FEAT))
    return out_pad[:, :N_CLASSES]


# speedup vs baseline: 11.7591x; 1.0459x over previous
import functools

import jax
import jax.numpy as jnp
from jax import lax
from jax.experimental import pallas as pl
from jax.experimental.pallas import tpu as pltpu
from jax.experimental.pallas import tpu_sc as plsc

N_NODES = 10000
N_EDGES = 320000
N_FEAT = 128
N_CLASSES = 40

NC = 2
NS = 16
NW = NC * NS

EPW = N_EDGES // NW
CH = 80
NCHUNK = EPW // CH
NBUF = 3
STAGE = 25
STAGES = NCHUNK // STAGE
PADR = 640
ACC_ROWS = NS * PADR

MB = 1000
NBLK = N_NODES // MB


def _agg_body(h_hbm, src_hbm, dst_hbm, z_hbm, out_hbm,
              src_v, dst_v, rows_v, acc_sh, gsem, ssem):
    c = lax.axis_index("c")
    s = lax.axis_index("s")
    wid = s * NC + c

    pltpu.sync_copy(z_hbm, rows_v.at[0])
    for k in range(PADR // CH):
        pltpu.sync_copy(rows_v.at[0], acc_sh.at[pl.ds(s * PADR + k * CH, CH)])
    rem = PADR - (PADR // CH) * CH
    if rem:
        pltpu.sync_copy(rows_v.at[0].at[pl.ds(0, rem)],
                        acc_sh.at[pl.ds(s * PADR + (PADR // CH) * CH, rem)])
    plsc.subcore_barrier()

    for st in range(STAGES):
        pltpu.sync_copy(src_hbm.at[wid, st], src_v)
        pltpu.sync_copy(dst_hbm.at[wid, st], dst_v)
        pltpu.async_copy(h_hbm.at[src_v.at[0]], rows_v.at[0], gsem)

        @pl.loop(0, STAGE)
        def _chunks(k):
            b = lax.rem(k, NBUF)
            nb = lax.rem(k + 1, NBUF)

            @pl.when(k >= NBUF - 1)
            def _drain_oldest():
                pltpu.make_async_copy(
                    rows_v.at[nb], acc_sh.at[dst_v.at[k - (NBUF - 1)]],
                    ssem).wait()

            @pl.when(k + 1 < STAGE)
            def _fire_next():
                pltpu.async_copy(h_hbm.at[src_v.at[k + 1]], rows_v.at[nb],
                                 gsem)

            pltpu.make_async_copy(h_hbm.at[src_v.at[k]], rows_v.at[b],
                                  gsem).wait()
            pltpu.async_copy(rows_v.at[b], acc_sh.at[dst_v.at[k]], ssem,
                             add=True)

        for d in range(NBUF - 1):
            pltpu.make_async_copy(rows_v.at[d],
                                  acc_sh.at[dst_v.at[STAGE - 1]], ssem).wait()

    plsc.subcore_barrier()

    pltpu.sync_copy(acc_sh.at[pl.ds(s * PADR, PADR)], out_hbm.at[c, s])


@jax.jit
def _aggregate(h, src3d, dst3d, zeros):
    mesh = plsc.VectorSubcoreMesh(core_axis_name="c", subcore_axis_name="s",
                                  num_cores=NC, num_subcores=NS)
    return pl.kernel(
        _agg_body,
        out_type=jax.ShapeDtypeStruct((NC, NS, PADR, N_FEAT), jnp.float32),
        mesh=mesh,
        scratch_types=[
            pltpu.VMEM((STAGE, CH), jnp.int32),
            pltpu.VMEM((STAGE, CH), jnp.int32),
            pltpu.VMEM((NBUF, CH, N_FEAT), jnp.float32),
            pltpu.VMEM_SHARED((ACC_ROWS, N_FEAT), jnp.float32),
            pltpu.SemaphoreType.DMA,
            pltpu.SemaphoreType.DMA,
        ],
    )(h, src3d, dst3d, zeros)


def _mm_body(x_ref, w_ref, b_ref, o_ref):
    o_ref[...] = jnp.dot(x_ref[...], w_ref[...],
                         preferred_element_type=jnp.float32) + b_ref[...]


def _fuse_body(p0_ref, p1_ref, w_ref, b_ref, h_ref, t_ref):
    h = jax.nn.relu(p0_ref[...] + p1_ref[...])
    h_ref[...] = h
    t_ref[...] = jnp.dot(h, w_ref[...],
                         preferred_element_type=jnp.float32) + b_ref[...]


def _fuse_res_body(p0_ref, p1_ref, r_ref, w_ref, b_ref, h_ref, t_ref):
    h = jax.nn.relu(p0_ref[...] + p1_ref[...]) + r_ref[...]
    h_ref[...] = h
    t_ref[...] = jnp.dot(h, w_ref[...],
                         preferred_element_type=jnp.float32) + b_ref[...]


def _final_body(p0_ref, p1_ref, r_ref, w_ref, b_ref, o_ref):
    h = jax.nn.relu(p0_ref[...] + p1_ref[...]) + r_ref[...]
    o_ref[...] = jnp.dot(h, w_ref[...],
                         preferred_element_type=jnp.float32) + b_ref[...]


_row_spec = pl.BlockSpec((MB, N_FEAT), lambda i: (i, 0))
_w_spec = pl.BlockSpec((N_FEAT, N_FEAT), lambda i: (0, 0))
_b_spec = pl.BlockSpec((1, N_FEAT), lambda i: (0, 0))


def _mm(x, w, b):
    return pl.pallas_call(
        _mm_body,
        grid=(NBLK,),
        in_specs=[_row_spec, _w_spec, _b_spec],
        out_specs=_row_spec,
        out_shape=jax.ShapeDtypeStruct((N_NODES, N_FEAT), jnp.float32),
    )(x, w, b)


def _fuse(p0, p1, w, b):
    return pl.pallas_call(
        _fuse_body,
        grid=(NBLK,),
        in_specs=[_row_spec, _row_spec, _w_spec, _b_spec],
        out_specs=[_row_spec, _row_spec],
        out_shape=[jax.ShapeDtypeStruct((N_NODES, N_FEAT), jnp.float32),
                   jax.ShapeDtypeStruct((N_NODES, N_FEAT), jnp.float32)],
    )(p0, p1, w, b)


def _fuse_res(p0, p1, r, w, b):
    return pl.pallas_call(
        _fuse_res_body,
        grid=(NBLK,),
        in_specs=[_row_spec, _row_spec, _row_spec, _w_spec, _b_spec],
        out_specs=[_row_spec, _row_spec],
        out_shape=[jax.ShapeDtypeStruct((N_NODES, N_FEAT), jnp.float32),
                   jax.ShapeDtypeStruct((N_NODES, N_FEAT), jnp.float32)],
    )(p0, p1, r, w, b)


def _final(p0, p1, r, w, b):
    return pl.pallas_call(
        _final_body,
        grid=(NBLK,),
        in_specs=[_row_spec, _row_spec, _row_spec, _w_spec, _b_spec],
        out_specs=_row_spec,
        out_shape=jax.ShapeDtypeStruct((N_NODES, N_FEAT), jnp.float32),
    )(p0, p1, r, w, b)


def kernel(x, edge_index, W0, b0, W1, b1, W2, b2, Wl, bl):
    src3d = edge_index[0].astype(jnp.int32).reshape(NW, STAGES, STAGE, CH)
    dst3d = edge_index[1].astype(jnp.int32).reshape(NW, STAGES, STAGE, CH)
    zeros = jnp.zeros((CH, N_FEAT), jnp.float32)
    wl_pad = jnp.zeros((N_FEAT, N_FEAT), jnp.float32).at[:, :N_CLASSES].set(Wl)
    bl_pad = jnp.zeros((N_FEAT,), jnp.float32).at[:N_CLASSES].set(bl)

    def parts(p):
        p = p.reshape(NC, ACC_ROWS, N_FEAT)
        return p[0, :N_NODES], p[1, :N_NODES]

    t0 = _mm(x, W0, b0.reshape(1, N_FEAT))
    p0, p1 = parts(_aggregate(t0, src3d, dst3d, zeros))
    h0, t1 = _fuse(p0, p1, W1, b1.reshape(1, N_FEAT))
    p0, p1 = parts(_aggregate(t1, src3d, dst3d, zeros))
    h1, t2 = _fuse_res(p0, p1, h0, W2, b2.reshape(1, N_FEAT))
    p0, p1 = parts(_aggregate(t2, src3d, dst3d, zeros))
    out_pad = _final(p0, p1, h1, wl_pad, bl_pad.reshape(1, N_FEAT))
    return out_pad[:, :N_CLASSES]


# speedup vs baseline: 12.2101x; 1.0384x over previous
import functools

import jax
import jax.numpy as jnp
from jax import lax
from jax.experimental import pallas as pl
from jax.experimental.pallas import tpu as pltpu
from jax.experimental.pallas import tpu_sc as plsc

N_NODES = 10000
N_EDGES = 320000
N_FEAT = 128
N_CLASSES = 40

NC = 2
NS = 16
NW = NC * NS

EPW = N_EDGES // NW
CH = 80
NCHUNK = EPW // CH
NBUF = 3
STAGE = 25
STAGES = NCHUNK // STAGE
PADR = 640
ACC_ROWS = NS * PADR

MB = 1000
NBLK = N_NODES // MB


def _agg_body(h_hbm, src_hbm, dst_hbm, z_hbm, out_hbm,
              src_v, dst_v, rows_v, acc_sh, gsem, ssem):
    c = lax.axis_index("c")
    s = lax.axis_index("s")
    wid = s * NC + c

    pltpu.sync_copy(z_hbm, rows_v.at[0])
    for k in range(PADR // CH):
        pltpu.sync_copy(rows_v.at[0], acc_sh.at[pl.ds(s * PADR + k * CH, CH)])
    rem = PADR - (PADR // CH) * CH
    if rem:
        pltpu.sync_copy(rows_v.at[0].at[pl.ds(0, rem)],
                        acc_sh.at[pl.ds(s * PADR + (PADR // CH) * CH, rem)])
    plsc.subcore_barrier()

    for st in range(STAGES):
        pltpu.sync_copy(src_hbm.at[wid, st], src_v)
        pltpu.sync_copy(dst_hbm.at[wid, st], dst_v)
        pltpu.async_copy(h_hbm.at[src_v.at[0]], rows_v.at[0], gsem)

        @pl.loop(0, STAGE)
        def _chunks(k):
            b = lax.rem(k, NBUF)
            nb = lax.rem(k + 1, NBUF)

            @pl.when(k >= NBUF - 1)
            def _drain_oldest():
                pltpu.make_async_copy(
                    rows_v.at[nb], acc_sh.at[dst_v.at[k - (NBUF - 1)]],
                    ssem).wait()

            @pl.when(k + 1 < STAGE)
            def _fire_next():
                pltpu.async_copy(h_hbm.at[src_v.at[k + 1]], rows_v.at[nb],
                                 gsem)

            pltpu.make_async_copy(h_hbm.at[src_v.at[k]], rows_v.at[b],
                                  gsem).wait()
            pltpu.async_copy(rows_v.at[b], acc_sh.at[dst_v.at[k]], ssem,
                             add=True)

        for d in range(NBUF - 1):
            pltpu.make_async_copy(rows_v.at[d],
                                  acc_sh.at[dst_v.at[STAGE - 1]], ssem).wait()

    plsc.subcore_barrier()

    pltpu.sync_copy(acc_sh.at[pl.ds(s * PADR, PADR)],
                    out_hbm.at[c, pl.ds(s * PADR, PADR)])


@jax.jit
def _aggregate(h, src3d, dst3d, zeros):
    mesh = plsc.VectorSubcoreMesh(core_axis_name="c", subcore_axis_name="s",
                                  num_cores=NC, num_subcores=NS)
    return pl.kernel(
        _agg_body,
        out_type=jax.ShapeDtypeStruct((NC, ACC_ROWS, N_FEAT), jnp.float32),
        mesh=mesh,
        scratch_types=[
            pltpu.VMEM((STAGE, CH), jnp.int32),
            pltpu.VMEM((STAGE, CH), jnp.int32),
            pltpu.VMEM((NBUF, CH, N_FEAT), jnp.float32),
            pltpu.VMEM_SHARED((ACC_ROWS, N_FEAT), jnp.float32),
            pltpu.SemaphoreType.DMA,
            pltpu.SemaphoreType.DMA,
        ],
    )(h, src3d, dst3d, zeros)


def _mm_body(x_ref, w_ref, b_ref, o_ref):
    o_ref[...] = jnp.dot(x_ref[...], w_ref[...],
                         preferred_element_type=jnp.float32) + b_ref[...]


def _fuse_body(p_ref, w_ref, b_ref, h_ref, t_ref):
    h = jax.nn.relu(p_ref[0] + p_ref[1])
    h_ref[...] = h
    t_ref[...] = jnp.dot(h, w_ref[...],
                         preferred_element_type=jnp.float32) + b_ref[...]


def _fuse_res_body(p_ref, r_ref, w_ref, b_ref, h_ref, t_ref):
    h = jax.nn.relu(p_ref[0] + p_ref[1]) + r_ref[...]
    h_ref[...] = h
    t_ref[...] = jnp.dot(h, w_ref[...],
                         preferred_element_type=jnp.float32) + b_ref[...]


def _final_body(p_ref, r_ref, w_ref, b_ref, o_ref):
    h = jax.nn.relu(p_ref[0] + p_ref[1]) + r_ref[...]
    o_ref[...] = jnp.dot(h, w_ref[...],
                         preferred_element_type=jnp.float32) + b_ref[...]


_row_spec = pl.BlockSpec((MB, N_FEAT), lambda i: (i, 0))
_p_spec = pl.BlockSpec((NC, MB, N_FEAT), lambda i: (0, i, 0))
_w_spec = pl.BlockSpec((N_FEAT, N_FEAT), lambda i: (0, 0))
_b_spec = pl.BlockSpec((1, N_FEAT), lambda i: (0, 0))
_wl_spec = pl.BlockSpec((N_FEAT, N_CLASSES), lambda i: (0, 0))
_bl_spec = pl.BlockSpec((1, N_CLASSES), lambda i: (0, 0))
_o_spec = pl.BlockSpec((MB, N_CLASSES), lambda i: (i, 0))
_hh = jax.ShapeDtypeStruct((N_NODES, N_FEAT), jnp.float32)


def _mm(x, w, b):
    return pl.pallas_call(
        _mm_body,
        grid=(NBLK,),
        in_specs=[_row_spec, _w_spec, _b_spec],
        out_specs=_row_spec,
        out_shape=_hh,
    )(x, w, b)


def _fuse(p, w, b):
    return pl.pallas_call(
        _fuse_body,
        grid=(NBLK,),
        in_specs=[_p_spec, _w_spec, _b_spec],
        out_specs=[_row_spec, _row_spec],
        out_shape=[_hh, _hh],
    )(p, w, b)


def _fuse_res(p, r, w, b):
    return pl.pallas_call(
        _fuse_res_body,
        grid=(NBLK,),
        in_specs=[_p_spec, _row_spec, _w_spec, _b_spec],
        out_specs=[_row_spec, _row_spec],
        out_shape=[_hh, _hh],
    )(p, r, w, b)


def _final(p, r, w, b):
    return pl.pallas_call(
        _final_body,
        grid=(NBLK,),
        in_specs=[_p_spec, _row_spec, _wl_spec, _bl_spec],
        out_specs=_o_spec,
        out_shape=jax.ShapeDtypeStruct((N_NODES, N_CLASSES), jnp.float32),
    )(p, r, w, b)


def kernel(x, edge_index, W0, b0, W1, b1, W2, b2, Wl, bl):
    src3d = edge_index[0].astype(jnp.int32).reshape(NW, STAGES, STAGE, CH)
    dst3d = edge_index[1].astype(jnp.int32).reshape(NW, STAGES, STAGE, CH)
    zeros = jnp.zeros((CH, N_FEAT), jnp.float32)

    t0 = _mm(x, W0, b0.reshape(1, N_FEAT))
    p = _aggregate(t0, src3d, dst3d, zeros)
    h0, t1 = _fuse(p, W1, b1.reshape(1, N_FEAT))
    p = _aggregate(t1, src3d, dst3d, zeros)
    h1, t2 = _fuse_res(p, h0, W2, b2.reshape(1, N_FEAT))
    p = _aggregate(t2, src3d, dst3d, zeros)
    return _final(p, h1, Wl, bl.reshape(1, N_CLASSES))


# speedup vs baseline: 13.0131x; 1.0658x over previous
import functools

import jax
import jax.numpy as jnp
from jax import lax
from jax.experimental import pallas as pl
from jax.experimental.pallas import tpu as pltpu
from jax.experimental.pallas import tpu_sc as plsc

N_NODES = 10000
N_EDGES = 320000
N_FEAT = 128
N_CLASSES = 40

NC = 2
NS = 16
NW = NC * NS

EPW = N_EDGES // NW
CH = 100
NCHUNK = EPW // CH
NBUF = 3
STAGE = 20
STAGES = NCHUNK // STAGE
PADR = 632
ACC_ROWS = NS * PADR

MB = 2000
NBLK = N_NODES // MB


def _agg_body(h_hbm, src_hbm, dst_hbm, z_hbm, out_hbm,
              src_v, dst_v, rows_v, acc_sh, gsem, ssem):
    c = lax.axis_index("c")
    s = lax.axis_index("s")
    wid = s * NC + c

    pltpu.sync_copy(z_hbm, rows_v.at[0])
    for k in range(PADR // CH):
        pltpu.sync_copy(rows_v.at[0], acc_sh.at[pl.ds(s * PADR + k * CH, CH)])
    rem = PADR - (PADR // CH) * CH
    if rem:
        pltpu.sync_copy(rows_v.at[0].at[pl.ds(0, rem)],
                        acc_sh.at[pl.ds(s * PADR + (PADR // CH) * CH, rem)])
    plsc.subcore_barrier()

    for st in range(STAGES):
        pltpu.sync_copy(src_hbm.at[wid, st], src_v)
        pltpu.sync_copy(dst_hbm.at[wid, st], dst_v)
        pltpu.async_copy(h_hbm.at[src_v.at[0]], rows_v.at[0], gsem)

        @pl.loop(0, STAGE)
        def _chunks(k):
            b = lax.rem(k, NBUF)
            nb = lax.rem(k + 1, NBUF)

            @pl.when(k >= NBUF - 1)
            def _drain_oldest():
                pltpu.make_async_copy(
                    rows_v.at[nb], acc_sh.at[dst_v.at[k - (NBUF - 1)]],
                    ssem).wait()

            @pl.when(k + 1 < STAGE)
            def _fire_next():
                pltpu.async_copy(h_hbm.at[src_v.at[k + 1]], rows_v.at[nb],
                                 gsem)

            pltpu.make_async_copy(h_hbm.at[src_v.at[k]], rows_v.at[b],
                                  gsem).wait()
            pltpu.async_copy(rows_v.at[b], acc_sh.at[dst_v.at[k]], ssem,
                             add=True)

        for d in range(NBUF - 1):
            pltpu.make_async_copy(rows_v.at[d],
                                  acc_sh.at[dst_v.at[STAGE - 1]], ssem).wait()

    plsc.subcore_barrier()

    pltpu.sync_copy(acc_sh.at[pl.ds(s * PADR, PADR)],
                    out_hbm.at[c, pl.ds(s * PADR, PADR)])


@jax.jit
def _aggregate(h, src3d, dst3d, zeros):
    mesh = plsc.VectorSubcoreMesh(core_axis_name="c", subcore_axis_name="s",
                                  num_cores=NC, num_subcores=NS)
    return pl.kernel(
        _agg_body,
        out_type=jax.ShapeDtypeStruct((NC, ACC_ROWS, N_FEAT), jnp.float32),
        mesh=mesh,
        scratch_types=[
            pltpu.VMEM((STAGE, CH), jnp.int32),
            pltpu.VMEM((STAGE, CH), jnp.int32),
            pltpu.VMEM((NBUF, CH, N_FEAT), jnp.float32),
            pltpu.VMEM_SHARED((ACC_ROWS, N_FEAT), jnp.float32),
            pltpu.SemaphoreType.DMA,
            pltpu.SemaphoreType.DMA,
        ],
    )(h, src3d, dst3d, zeros)


def _mm_body(x_ref, w_ref, b_ref, o_ref):
    o_ref[...] = jnp.dot(x_ref[...], w_ref[...],
                         preferred_element_type=jnp.float32) + b_ref[...]


def _fuse_body(p_ref, w_ref, b_ref, h_ref, t_ref):
    h = jax.nn.relu(p_ref[0] + p_ref[1])
    h_ref[...] = h
    t_ref[...] = jnp.dot(h, w_ref[...],
                         preferred_element_type=jnp.float32) + b_ref[...]


def _fuse_res_body(p_ref, r_ref, w_ref, b_ref, h_ref, t_ref):
    h = jax.nn.relu(p_ref[0] + p_ref[1]) + r_ref[...]
    h_ref[...] = h
    t_ref[...] = jnp.dot(h, w_ref[...],
                         preferred_element_type=jnp.float32) + b_ref[...]


def _final_body(p_ref, r_ref, w_ref, b_ref, o_ref):
    h = jax.nn.relu(p_ref[0] + p_ref[1]) + r_ref[...]
    o_ref[...] = jnp.dot(h, w_ref[...],
                         preferred_element_type=jnp.float32) + b_ref[...]


_row_spec = pl.BlockSpec((MB, N_FEAT), lambda i: (i, 0))
_p_spec = pl.BlockSpec((NC, MB, N_FEAT), lambda i: (0, i, 0))
_w_spec = pl.BlockSpec((N_FEAT, N_FEAT), lambda i: (0, 0))
_b_spec = pl.BlockSpec((1, N_FEAT), lambda i: (0, 0))
_wl_spec = pl.BlockSpec((N_FEAT, N_CLASSES), lambda i: (0, 0))
_bl_spec = pl.BlockSpec((1, N_CLASSES), lambda i: (0, 0))
_o_spec = pl.BlockSpec((MB, N_CLASSES), lambda i: (i, 0))
_hh = jax.ShapeDtypeStruct((N_NODES, N_FEAT), jnp.float32)


def _mm(x, w, b):
    return pl.pallas_call(
        _mm_body,
        grid=(NBLK,),
        in_specs=[_row_spec, _w_spec, _b_spec],
        out_specs=_row_spec,
        out_shape=_hh,
    )(x, w, b)


def _fuse(p, w, b):
    return pl.pallas_call(
        _fuse_body,
        grid=(NBLK,),
        in_specs=[_p_spec, _w_spec, _b_spec],
        out_specs=[_row_spec, _row_spec],
        out_shape=[_hh, _hh],
    )(p, w, b)


def _fuse_res(p, r, w, b):
    return pl.pallas_call(
        _fuse_res_body,
        grid=(NBLK,),
        in_specs=[_p_spec, _row_spec, _w_spec, _b_spec],
        out_specs=[_row_spec, _row_spec],
        out_shape=[_hh, _hh],
    )(p, r, w, b)


def _final(p, r, w, b):
    return pl.pallas_call(
        _final_body,
        grid=(NBLK,),
        in_specs=[_p_spec, _row_spec, _wl_spec, _bl_spec],
        out_specs=_o_spec,
        out_shape=jax.ShapeDtypeStruct((N_NODES, N_CLASSES), jnp.float32),
    )(p, r, w, b)


def kernel(x, edge_index, W0, b0, W1, b1, W2, b2, Wl, bl):
    src3d = edge_index[0].astype(jnp.int32).reshape(NW, STAGES, STAGE, CH)
    dst3d = edge_index[1].astype(jnp.int32).reshape(NW, STAGES, STAGE, CH)
    zeros = jnp.zeros((CH, N_FEAT), jnp.float32)

    t0 = _mm(x, W0, b0.reshape(1, N_FEAT))
    p = _aggregate(t0, src3d, dst3d, zeros)
    h0, t1 = _fuse(p, W1, b1.reshape(1, N_FEAT))
    p = _aggregate(t1, src3d, dst3d, zeros)
    h1, t2 = _fuse_res(p, h0, W2, b2.reshape(1, N_FEAT))
    p = _aggregate(t2, src3d, dst3d, zeros)
    return _final(p, h1, Wl, bl.reshape(1, N_CLASSES))


# speedup vs baseline: 13.3743x; 1.0278x over previous
import functools

import jax
import jax.numpy as jnp
from jax import lax
from jax.experimental import pallas as pl
from jax.experimental.pallas import tpu as pltpu
from jax.experimental.pallas import tpu_sc as plsc

N_NODES = 10000
N_EDGES = 320000
N_FEAT = 128
N_CLASSES = 40

NC = 2
NS = 16
NW = NC * NS

EPW = N_EDGES // NW
CH = 100
NCHUNK = EPW // CH
NBUF = 3
STAGE = 20
STAGES = NCHUNK // STAGE
PADR = 632
ACC_ROWS = NS * PADR

MB = 2000
NBLK = N_NODES // MB


def _agg_body(h_hbm, edge_hbm, z_hbm, out_hbm,
              src_v, dst_v, rows_v, acc_sh, gsem, ssem):
    c = lax.axis_index("c")
    s = lax.axis_index("s")
    wid = s * NC + c

    pltpu.sync_copy(z_hbm, rows_v.at[0])
    for k in range(PADR // CH):
        pltpu.sync_copy(rows_v.at[0], acc_sh.at[pl.ds(s * PADR + k * CH, CH)])
    rem = PADR - (PADR // CH) * CH
    if rem:
        pltpu.sync_copy(rows_v.at[0].at[pl.ds(0, rem)],
                        acc_sh.at[pl.ds(s * PADR + (PADR // CH) * CH, rem)])
    plsc.subcore_barrier()

    for st in range(STAGES):
        pltpu.sync_copy(edge_hbm.at[0, wid, st], src_v)
        pltpu.sync_copy(edge_hbm.at[1, wid, st], dst_v)
        pltpu.async_copy(h_hbm.at[src_v.at[0]], rows_v.at[0], gsem)

        @pl.loop(0, STAGE)
        def _chunks(k):
            b = lax.rem(k, NBUF)
            nb = lax.rem(k + 1, NBUF)

            @pl.when(k >= NBUF - 1)
            def _drain_oldest():
                pltpu.make_async_copy(
                    rows_v.at[nb], acc_sh.at[dst_v.at[k - (NBUF - 1)]],
                    ssem).wait()

            @pl.when(k + 1 < STAGE)
            def _fire_next():
                pltpu.async_copy(h_hbm.at[src_v.at[k + 1]], rows_v.at[nb],
                                 gsem)

            pltpu.make_async_copy(h_hbm.at[src_v.at[k]], rows_v.at[b],
                                  gsem).wait()
            pltpu.async_copy(rows_v.at[b], acc_sh.at[dst_v.at[k]], ssem,
                             add=True)

        for d in range(NBUF - 1):
            pltpu.make_async_copy(rows_v.at[d],
                                  acc_sh.at[dst_v.at[STAGE - 1]], ssem).wait()

    plsc.subcore_barrier()

    pltpu.sync_copy(acc_sh.at[pl.ds(s * PADR, PADR)],
                    out_hbm.at[c, pl.ds(s * PADR, PADR)])


@jax.jit
def _aggregate(h, edges, zeros):
    mesh = plsc.VectorSubcoreMesh(core_axis_name="c", subcore_axis_name="s",
                                  num_cores=NC, num_subcores=NS)
    return pl.kernel(
        _agg_body,
        out_type=jax.ShapeDtypeStruct((NC, ACC_ROWS, N_FEAT), jnp.float32),
        mesh=mesh,
        scratch_types=[
            pltpu.VMEM((STAGE, CH), jnp.int32),
            pltpu.VMEM((STAGE, CH), jnp.int32),
            pltpu.VMEM((NBUF, CH, N_FEAT), jnp.float32),
            pltpu.VMEM_SHARED((ACC_ROWS, N_FEAT), jnp.float32),
            pltpu.SemaphoreType.DMA,
            pltpu.SemaphoreType.DMA,
        ],
    )(h, edges, zeros)


def _mm_body(x_ref, w_ref, b_ref, o_ref):
    o_ref[...] = jnp.dot(x_ref[...], w_ref[...],
                         preferred_element_type=jnp.float32) + b_ref[...]


def _fuse_body(p_ref, w_ref, b_ref, h_ref, t_ref):
    h = jax.nn.relu(p_ref[0] + p_ref[1])
    h_ref[...] = h
    t_ref[...] = jnp.dot(h, w_ref[...],
                         preferred_element_type=jnp.float32) + b_ref[...]


def _fuse_res_body(p_ref, r_ref, w_ref, b_ref, h_ref, t_ref):
    h = jax.nn.relu(p_ref[0] + p_ref[1]) + r_ref[...]
    h_ref[...] = h
    t_ref[...] = jnp.dot(h, w_ref[...],
                         preferred_element_type=jnp.float32) + b_ref[...]


def _final_body(p_ref, r_ref, w_ref, b_ref, o_ref):
    h = jax.nn.relu(p_ref[0] + p_ref[1]) + r_ref[...]
    o_ref[...] = jnp.dot(h, w_ref[...],
                         preferred_element_type=jnp.float32) + b_ref[...]


_row_spec = pl.BlockSpec((MB, N_FEAT), lambda i: (i, 0))
_p_spec = pl.BlockSpec((NC, MB, N_FEAT), lambda i: (0, i, 0))
_w_spec = pl.BlockSpec((N_FEAT, N_FEAT), lambda i: (0, 0))
_b_spec = pl.BlockSpec((1, N_FEAT), lambda i: (0, 0))
_wl_spec = pl.BlockSpec((N_FEAT, N_CLASSES), lambda i: (0, 0))
_bl_spec = pl.BlockSpec((1, N_CLASSES), lambda i: (0, 0))
_o_spec = pl.BlockSpec((MB, N_CLASSES), lambda i: (i, 0))
_hh = jax.ShapeDtypeStruct((N_NODES, N_FEAT), jnp.float32)


def _mm(x, w, b):
    return pl.pallas_call(
        _mm_body,
        grid=(NBLK,),
        in_specs=[_row_spec, _w_spec, _b_spec],
        out_specs=_row_spec,
        out_shape=_hh,
    )(x, w, b)


def _fuse(p, w, b):
    return pl.pallas_call(
        _fuse_body,
        grid=(NBLK,),
        in_specs=[_p_spec, _w_spec, _b_spec],
        out_specs=[_row_spec, _row_spec],
        out_shape=[_hh, _hh],
    )(p, w, b)


def _fuse_res(p, r, w, b):
    return pl.pallas_call(
        _fuse_res_body,
        grid=(NBLK,),
        in_specs=[_p_spec, _row_spec, _w_spec, _b_spec],
        out_specs=[_row_spec, _row_spec],
        out_shape=[_hh, _hh],
    )(p, r, w, b)


def _final(p, r, w, b):
    return pl.pallas_call(
        _final_body,
        grid=(NBLK,),
        in_specs=[_p_spec, _row_spec, _wl_spec, _bl_spec],
        out_specs=_o_spec,
        out_shape=jax.ShapeDtypeStruct((N_NODES, N_CLASSES), jnp.float32),
    )(p, r, w, b)


def kernel(x, edge_index, W0, b0, W1, b1, W2, b2, Wl, bl):
    edges = edge_index.astype(jnp.int32).reshape(2, NW, STAGES, STAGE, CH)
    zeros = jnp.zeros((CH, N_FEAT), jnp.float32)

    t0 = _mm(x, W0, b0.reshape(1, N_FEAT))
    p = _aggregate(t0, edges, zeros)
    h0, t1 = _fuse(p, W1, b1.reshape(1, N_FEAT))
    p = _aggregate(t1, edges, zeros)
    h1, t2 = _fuse_res(p, h0, W2, b2.reshape(1, N_FEAT))
    p = _aggregate(t2, edges, zeros)
    return _final(p, h1, Wl, bl.reshape(1, N_CLASSES))


# speedup vs baseline: 14.1079x; 1.0549x over previous
import functools

import jax
import jax.numpy as jnp
from jax import lax
from jax.experimental import pallas as pl
from jax.experimental.pallas import tpu as pltpu
from jax.experimental.pallas import tpu_sc as plsc

N_NODES = 10000
N_EDGES = 320000
N_FEAT = 128
N_CLASSES = 40

NC = 2
NS = 16
NW = NC * NS

EPW = N_EDGES // NW
CH = 100
NCHUNK = EPW // CH
NBUF = 3
STAGE = 10
STAGES = NCHUNK // STAGE
PADR = 632
ACC_ROWS = NS * PADR

MB = 2000
NBLK = N_NODES // MB


def _agg_body(h_hbm, edge_hbm, z_hbm, out_hbm,
              src_v, dst_v, rows_v, acc_sh, gsem, ssem, isem):
    c = lax.axis_index("c")
    s = lax.axis_index("s")
    wid = s * NC + c

    pltpu.sync_copy(z_hbm, rows_v.at[0])
    for k in range(PADR // CH):
        pltpu.sync_copy(rows_v.at[0], acc_sh.at[pl.ds(s * PADR + k * CH, CH)])
    rem = PADR - (PADR // CH) * CH
    if rem:
        pltpu.sync_copy(rows_v.at[0].at[pl.ds(0, rem)],
                        acc_sh.at[pl.ds(s * PADR + (PADR // CH) * CH, rem)])
    plsc.subcore_barrier()

    pltpu.sync_copy(edge_hbm.at[0, wid, 0], src_v.at[0])
    pltpu.sync_copy(edge_hbm.at[1, wid, 0], dst_v.at[0])
    pltpu.async_copy(h_hbm.at[src_v.at[0, 0]], rows_v.at[0], gsem)

    @pl.loop(0, NCHUNK)
    def _chunks(k):
        s_cur = lax.div(k, STAGE)
        bi = lax.rem(s_cur, 2)
        r = k - s_cur * STAGE
        kn = k + 1
        sn = lax.div(kn, STAGE)
        bin_ = lax.rem(sn, 2)
        rn = kn - sn * STAGE
        b = lax.rem(k, NBUF)
        nb = lax.rem(kn, NBUF)

        @pl.when(jnp.logical_and(r == 2, s_cur + 1 < STAGES))
        def _prefetch_idx():
            sp = s_cur + 1
            bp = lax.rem(sp, 2)
            pltpu.async_copy(edge_hbm.at[0, wid, sp], src_v.at[bp], isem)
            pltpu.async_copy(edge_hbm.at[1, wid, sp], dst_v.at[bp], isem)

        @pl.when(k >= NBUF - 1)
        def _drain_oldest():
            pltpu.make_async_copy(rows_v.at[nb], acc_sh.at[dst_v.at[bi, r]],
                                  ssem).wait()

        @pl.when(jnp.logical_and(rn == 0, kn < NCHUNK))
        def _await_idx():
            pltpu.make_async_copy(edge_hbm.at[0, wid, 0], src_v.at[0],
                                  isem).wait()
            pltpu.make_async_copy(edge_hbm.at[1, wid, 0], dst_v.at[0],
                                  isem).wait()

        @pl.when(kn < NCHUNK)
        def _fire_next():
            pltpu.async_copy(h_hbm.at[src_v.at[bin_, rn]], rows_v.at[nb],
                             gsem)

        pltpu.make_async_copy(h_hbm.at[src_v.at[bi, r]], rows_v.at[b],
                              gsem).wait()
        pltpu.async_copy(rows_v.at[b], acc_sh.at[dst_v.at[bi, r]], ssem,
                         add=True)

    for d in range(NBUF - 1):
        pltpu.make_async_copy(rows_v.at[d], acc_sh.at[dst_v.at[0, 0]],
                              ssem).wait()

    plsc.subcore_barrier()

    pltpu.sync_copy(acc_sh.at[pl.ds(s * PADR, PADR)],
                    out_hbm.at[c, pl.ds(s * PADR, PADR)])


@jax.jit
def _aggregate(h, edges, zeros):
    mesh = plsc.VectorSubcoreMesh(core_axis_name="c", subcore_axis_name="s",
                                  num_cores=NC, num_subcores=NS)
    return pl.kernel(
        _agg_body,
        out_type=jax.ShapeDtypeStruct((NC, ACC_ROWS, N_FEAT), jnp.float32),
        mesh=mesh,
        scratch_types=[
            pltpu.VMEM((2, STAGE, CH), jnp.int32),
            pltpu.VMEM((2, STAGE, CH), jnp.int32),
            pltpu.VMEM((NBUF, CH, N_FEAT), jnp.float32),
            pltpu.VMEM_SHARED((ACC_ROWS, N_FEAT), jnp.float32),
            pltpu.SemaphoreType.DMA,
            pltpu.SemaphoreType.DMA,
            pltpu.SemaphoreType.DMA,
        ],
    )(h, edges, zeros)


def _mm_body(x_ref, w_ref, b_ref, o_ref):
    o_ref[...] = jnp.dot(x_ref[...], w_ref[...],
                         preferred_element_type=jnp.float32) + b_ref[...]


def _fuse_body(p_ref, w_ref, b_ref, h_ref, t_ref):
    h = jax.nn.relu(p_ref[0] + p_ref[1])
    h_ref[...] = h
    t_ref[...] = jnp.dot(h, w_ref[...],
                         preferred_element_type=jnp.float32) + b_ref[...]


def _fuse_res_body(p_ref, r_ref, w_ref, b_ref, h_ref, t_ref):
    h = jax.nn.relu(p_ref[0] + p_ref[1]) + r_ref[...]
    h_ref[...] = h
    t_ref[...] = jnp.dot(h, w_ref[...],
                         preferred_element_type=jnp.float32) + b_ref[...]


def _final_body(p_ref, r_ref, w_ref, b_ref, o_ref):
    h = jax.nn.relu(p_ref[0] + p_ref[1]) + r_ref[...]
    o_ref[...] = jnp.dot(h, w_ref[...],
                         preferred_element_type=jnp.float32) + b_ref[...]


_row_spec = pl.BlockSpec((MB, N_FEAT), lambda i: (i, 0))
_p_spec = pl.BlockSpec((NC, MB, N_FEAT), lambda i: (0, i, 0))
_w_spec = pl.BlockSpec((N_FEAT, N_FEAT), lambda i: (0, 0))
_b_spec = pl.BlockSpec((1, N_FEAT), lambda i: (0, 0))
_wl_spec = pl.BlockSpec((N_FEAT, N_CLASSES), lambda i: (0, 0))
_bl_spec = pl.BlockSpec((1, N_CLASSES), lambda i: (0, 0))
_o_spec = pl.BlockSpec((MB, N_CLASSES), lambda i: (i, 0))
_hh = jax.ShapeDtypeStruct((N_NODES, N_FEAT), jnp.float32)


def _mm(x, w, b):
    return pl.pallas_call(
        _mm_body,
        grid=(NBLK,),
        in_specs=[_row_spec, _w_spec, _b_spec],
        out_specs=_row_spec,
        out_shape=_hh,
    )(x, w, b)


def _fuse(p, w, b):
    return pl.pallas_call(
        _fuse_body,
        grid=(NBLK,),
        in_specs=[_p_spec, _w_spec, _b_spec],
        out_specs=[_row_spec, _row_spec],
        out_shape=[_hh, _hh],
    )(p, w, b)


def _fuse_res(p, r, w, b):
    return pl.pallas_call(
        _fuse_res_body,
        grid=(NBLK,),
        in_specs=[_p_spec, _row_spec, _w_spec, _b_spec],
        out_specs=[_row_spec, _row_spec],
        out_shape=[_hh, _hh],
    )(p, r, w, b)


def _final(p, r, w, b):
    return pl.pallas_call(
        _final_body,
        grid=(NBLK,),
        in_specs=[_p_spec, _row_spec, _wl_spec, _bl_spec],
        out_specs=_o_spec,
        out_shape=jax.ShapeDtypeStruct((N_NODES, N_CLASSES), jnp.float32),
    )(p, r, w, b)


def kernel(x, edge_index, W0, b0, W1, b1, W2, b2, Wl, bl):
    edges = edge_index.astype(jnp.int32).reshape(2, NW, STAGES, STAGE, CH)
    zeros = jnp.zeros((CH, N_FEAT), jnp.float32)

    t0 = _mm(x, W0, b0.reshape(1, N_FEAT))
    p = _aggregate(t0, edges, zeros)
    h0, t1 = _fuse(p, W1, b1.reshape(1, N_FEAT))
    p = _aggregate(t1, edges, zeros)
    h1, t2 = _fuse_res(p, h0, W2, b2.reshape(1, N_FEAT))
    p = _aggregate(t2, edges, zeros)
    return _final(p, h1, Wl, bl.reshape(1, N_CLASSES))
